# baseline (speedup 1.0000x reference)
import functools

import jax
import jax.numpy as jnp
from jax import lax
from jax.experimental import pallas as pl
from jax.experimental.pallas import tpu as pltpu
from jax.experimental.pallas import tpu_sc as plsc

D = 256
V = 8192
N = 9216
TI = 512
TJ = 4096
NI = N // TI
NJ = V // TJ


NSTEPS = NI * NJ
ROWS = TJ // 8


def _epilogue(mm_ref, xsq_ref, wsq_ref, pj, bval, bidx):
    xb = jnp.broadcast_to(xsq_ref[...], (8, TI))
    bv = None
    bi = None
    for r in range(ROWS):
        mm_r = mm_ref[pl.ds(r * 8, 8), :]
        w_r = jnp.broadcast_to(wsq_ref[pl.ds(r * 8, 8), :], (8, TI))
        d = (xb + w_r) - mm_r
        ridx = jnp.full((8, TI), r, jnp.int32)
        if r == 0:
            bv, bi = d, ridx
        else:
            lt = d < bv
            bv = jnp.where(lt, d, bv)
            bi = jnp.where(lt, ridx, bi)
    sub = lax.broadcasted_iota(jnp.int32, (8, TI), 0)
    bj = bi * 8 + sub
    v, jx = bv, bj
    for h in (4, 2, 1):
        v0, v1 = v[:h], v[h : 2 * h]
        j0, j1 = jx[:h], jx[h : 2 * h]
        take1 = (v1 < v0) | ((v1 == v0) & (j1 < j0))
        v = jnp.where(take1, v1, v0)
        jx = jnp.where(take1, j1, j0)
    gidx = jx + pj * TJ

    better = (v < bval[...]) | (pj == 0)
    bval[...] = jnp.where(better, v, bval[...])
    bidx[...] = jnp.where(better, gidx, bidx[...])


def _dot_tile(w_ref, j, x_ref):
    w_tile = w_ref[pl.ds(j * TJ, TJ), :]
    return lax.dot_general(
        w_tile, x_ref[...], (((1,), (1,)), ((), ())),
        preferred_element_type=jnp.float32,
        precision=lax.Precision.DEFAULT,
    )


def _argmin_body(x_ref, w_ref, xsqa_ref, wsqa_ref, xsqb_ref, wsqb_ref,
                 out_ref, mma_ref, mmb_ref, bval, bidx):
    t = pl.program_id(0)
    c = NSTEPS - 1
    ja = jnp.minimum(2 * t, c) % NJ
    jb = jnp.minimum(2 * t + 1, c) % NJ
    pja = jnp.maximum(2 * t - 1, 0) % NJ
    pjb = jnp.minimum(2 * t, c) % NJ

    mma_ref[...] = _dot_tile(w_ref, ja, x_ref)
    _epilogue(mmb_ref, xsqa_ref, wsqa_ref, pja, bval, bidx)
    out_ref[0] = bidx[...]
    mmb_ref[...] = _dot_tile(w_ref, jb, x_ref)
    _epilogue(mma_ref, xsqb_ref, wsqb_ref, pjb, bval, bidx)


def _bmu_indices(flat, emb_weight, xsq_row, wsq_col, i0, ni):
    nsteps = ni * NJ
    c = nsteps - 1
    out = pl.pallas_call(
        _argmin_body,
        grid=(nsteps // 2 + 1,),
        in_specs=[
            pl.BlockSpec(
                (TI, D), lambda t: (i0 + jnp.minimum(2 * t, c) // NJ, 0)),
            pl.BlockSpec((V, D), lambda t: (0, 0)),
            pl.BlockSpec(
                (1, TI), lambda t: (0, i0 + jnp.maximum(2 * t - 1, 0) // NJ)),
            pl.BlockSpec((TJ, 1), lambda t: (jnp.maximum(2 * t - 1, 0) % NJ, 0)),
            pl.BlockSpec(
                (1, TI), lambda t: (0, i0 + jnp.minimum(2 * t, c) // NJ)),
            pl.BlockSpec((TJ, 1), lambda t: (jnp.minimum(2 * t, c) % NJ, 0)),
        ],
        out_specs=pl.BlockSpec(
            (1, 1, TI), lambda t: (jnp.maximum(2 * t - 1, 0) // NJ, 0, 0)),
        out_shape=jax.ShapeDtypeStruct((ni, 1, TI), jnp.int32),
        scratch_shapes=[
            pltpu.VMEM((TJ, TI), jnp.float32),
            pltpu.VMEM((TJ, TI), jnp.float32),
            pltpu.VMEM((1, TI), jnp.float32),
            pltpu.VMEM((1, TI), jnp.int32),
        ],
    )(flat, emb_weight, xsq_row, wsq_col, xsq_row, wsq_col)
    return out.reshape(ni * TI)


def _sc_gather(table, idx, rows):
    NW = 32
    b_per_w = rows // NW
    mesh = plsc.VectorSubcoreMesh(core_axis_name="c", subcore_axis_name="s")

    @functools.partial(
        pl.kernel, mesh=mesh,
        out_type=jax.ShapeDtypeStruct((rows, D), jnp.float32),
        scratch_types=[
            pltpu.VMEM((b_per_w,), jnp.int32),
            pltpu.VMEM((b_per_w, D), jnp.float32),
            pltpu.SemaphoreType.DMA,
        ],
    )
    def k(table_hbm, idx_hbm, out_hbm, idx_v, rows_v, sem):
        wid = lax.axis_index("s") * 2 + lax.axis_index("c")
        base = wid * b_per_w
        pltpu.sync_copy(idx_hbm.at[pl.ds(base, b_per_w)], idx_v)
        pltpu.async_copy(table_hbm.at[idx_v], rows_v, sem).wait()
        pltpu.sync_copy(rows_v, out_hbm.at[pl.ds(base, b_per_w)])

    return k(table, idx)


def kernel(input, emb_weight):
    flat = input.reshape(-1, D)
    xsq = jnp.sum(flat ** 2, axis=1) * 0.5
    wsq = jnp.sum(emb_weight ** 2, axis=1) * 0.5
    xsq_row = xsq.reshape(1, N)
    wsq_col = wsq.reshape(V, 1)
    nh = NI // 2
    h = nh * TI
    idx1 = _bmu_indices(flat, emb_weight, xsq_row, wsq_col, 0, nh)
    out1 = _sc_gather(emb_weight, idx1, h)
    idx2 = _bmu_indices(flat, emb_weight, xsq_row, wsq_col, nh, NI - nh)
    out2 = _sc_gather(emb_weight, idx2, N - h)
    out_flat = jnp.concatenate([out1, out2], axis=0)
    return out_flat.reshape(input.shape)


# speedup vs baseline: 1.1631x; 1.1631x over previous
import functools

import jax
import jax.numpy as jnp
from jax import lax
from jax.experimental import pallas as pl
from jax.experimental.pallas import tpu as pltpu
from jax.experimental.pallas import tpu_sc as plsc

D = 256
V = 8192
N = 9216
TI = 512
TJ = 4096
NI = N // TI
NJ = V // TJ


NSTEPS = NI * NJ
ROWS = TJ // 8


def _epilogue(mm_ref, xsq_ref, wsq_ref, pj, bval, bidx):
    xb = jnp.broadcast_to(xsq_ref[...], (8, TI))
    bv = None
    bi = None
    for r in range(ROWS):
        mm_r = mm_ref[pl.ds(r * 8, 8), :]
        w_r = jnp.broadcast_to(wsq_ref[pl.ds(r * 8, 8), :], (8, TI))
        d = (xb + w_r) - mm_r
        ridx = jnp.full((8, TI), r, jnp.int32)
        if r == 0:
            bv, bi = d, ridx
        else:
            lt = d < bv
            bv = jnp.where(lt, d, bv)
            bi = jnp.where(lt, ridx, bi)
    sub = lax.broadcasted_iota(jnp.int32, (8, TI), 0)
    bj = bi * 8 + sub
    v, jx = bv, bj
    for h in (4, 2, 1):
        v0, v1 = v[:h], v[h : 2 * h]
        j0, j1 = jx[:h], jx[h : 2 * h]
        take1 = (v1 < v0) | ((v1 == v0) & (j1 < j0))
        v = jnp.where(take1, v1, v0)
        jx = jnp.where(take1, j1, j0)
    gidx = jx + pj * TJ

    better = (v < bval[...]) | (pj == 0)
    bval[...] = jnp.where(better, v, bval[...])
    bidx[...] = jnp.where(better, gidx, bidx[...])


def _dot_tile(w_ref, j, x_ref):
    w_tile = w_ref[pl.ds(j * TJ, TJ), :]
    return lax.dot_general(
        w_tile, x_ref[...], (((1,), (1,)), ((), ())),
        preferred_element_type=jnp.float32,
        precision=lax.Precision.DEFAULT,
    )


def _argmin_body(x_ref, w_ref, xsqa_ref, wsqa_ref, xsqb_ref, wsqb_ref,
                 out_ref, mma_ref, mmb_ref, bval, bidx):
    t = pl.program_id(0)
    c = NSTEPS - 1
    ja = jnp.minimum(2 * t, c) % NJ
    jb = jnp.minimum(2 * t + 1, c) % NJ
    pja = jnp.maximum(2 * t - 1, 0) % NJ
    pjb = jnp.minimum(2 * t, c) % NJ

    mma_ref[...] = _dot_tile(w_ref, ja, x_ref)
    _epilogue(mmb_ref, xsqa_ref, wsqa_ref, pja, bval, bidx)
    out_ref[0] = bidx[...]
    mmb_ref[...] = _dot_tile(w_ref, jb, x_ref)
    _epilogue(mma_ref, xsqb_ref, wsqb_ref, pjb, bval, bidx)


def _bmu_indices(flat, emb_weight, xsq_row, wsq_col):
    c = NSTEPS - 1
    out = pl.pallas_call(
        _argmin_body,
        grid=(NSTEPS // 2 + 1,),
        in_specs=[
            pl.BlockSpec((TI, D), lambda t: (jnp.minimum(2 * t, c) // NJ, 0)),
            pl.BlockSpec((V, D), lambda t: (0, 0)),
            pl.BlockSpec((1, TI), lambda t: (0, jnp.maximum(2 * t - 1, 0) // NJ)),
            pl.BlockSpec((TJ, 1), lambda t: (jnp.maximum(2 * t - 1, 0) % NJ, 0)),
            pl.BlockSpec((1, TI), lambda t: (0, jnp.minimum(2 * t, c) // NJ)),
            pl.BlockSpec((TJ, 1), lambda t: (jnp.minimum(2 * t, c) % NJ, 0)),
        ],
        out_specs=pl.BlockSpec(
            (1, 1, TI), lambda t: (jnp.maximum(2 * t - 1, 0) // NJ, 0, 0)),
        out_shape=jax.ShapeDtypeStruct((NI, 1, TI), jnp.int32),
        scratch_shapes=[
            pltpu.VMEM((TJ, TI), jnp.float32),
            pltpu.VMEM((TJ, TI), jnp.float32),
            pltpu.VMEM((1, TI), jnp.float32),
            pltpu.VMEM((1, TI), jnp.int32),
        ],
    )(flat, emb_weight, xsq_row, wsq_col, xsq_row, wsq_col)
    return out.reshape(N)


def _sc_gather(table, idx):
    NW = 32
    b_per_w = N // NW
    mesh = plsc.VectorSubcoreMesh(core_axis_name="c", subcore_axis_name="s")

    CH = 3
    rc = b_per_w // CH

    @functools.partial(
        pl.kernel, mesh=mesh,
        out_type=jax.ShapeDtypeStruct((N, D), jnp.float32),
        scratch_types=[
            pltpu.VMEM((b_per_w,), jnp.int32),
            pltpu.VMEM((rc, D), jnp.float32),
            pltpu.VMEM((rc, D), jnp.float32),
            pltpu.SemaphoreType.DMA,
            pltpu.SemaphoreType.DMA,
        ],
    )
    def k(table_hbm, idx_hbm, out_hbm, idx_v, rows0, rows1, sem0, sem1):
        wid = lax.axis_index("s") * 2 + lax.axis_index("c")
        base = wid * b_per_w
        pltpu.sync_copy(idx_hbm.at[pl.ds(base, b_per_w)], idx_v)
        g0 = pltpu.async_copy(table_hbm.at[idx_v.at[pl.ds(0, rc)]], rows0, sem0)
        g1 = pltpu.async_copy(table_hbm.at[idx_v.at[pl.ds(rc, rc)]], rows1, sem1)
        g0.wait()
        pltpu.sync_copy(rows0, out_hbm.at[pl.ds(base, rc)])
        g2 = pltpu.async_copy(table_hbm.at[idx_v.at[pl.ds(2 * rc, rc)]], rows0, sem0)
        g1.wait()
        pltpu.sync_copy(rows1, out_hbm.at[pl.ds(base + rc, rc)])
        g2.wait()
        pltpu.sync_copy(rows0, out_hbm.at[pl.ds(base + 2 * rc, rc)])

    return k(table, idx)


def kernel(input, emb_weight):
    flat = input.reshape(-1, D)
    xsq = jnp.sum(flat ** 2, axis=1) * 0.5
    wsq = jnp.sum(emb_weight ** 2, axis=1) * 0.5
    idx = _bmu_indices(flat, emb_weight, xsq.reshape(1, N), wsq.reshape(V, 1))
    out_flat = _sc_gather(emb_weight, idx)
    return out_flat.reshape(input.shape)


# speedup vs baseline: 1.1755x; 1.0107x over previous
import functools

import jax
import jax.numpy as jnp
from jax import lax
from jax.experimental import pallas as pl
from jax.experimental.pallas import tpu as pltpu
from jax.experimental.pallas import tpu_sc as plsc

D = 256
V = 8192
N = 9216
TI = 512
TJ = 4096
NI = N // TI
NJ = V // TJ


NSTEPS = NI * NJ
ROWS = TJ // 8


def _epilogue(mm_ref, xsq_ref, wsq_ref, pj, bval, bidx):
    xb = jnp.broadcast_to(xsq_ref[...], (8, TI))
    bv = None
    bi = None
    for r in range(ROWS):
        mm_r = mm_ref[pl.ds(r * 8, 8), :]
        w_r = jnp.broadcast_to(wsq_ref[pl.ds(r * 8, 8), :], (8, TI))
        d = (xb + w_r) - mm_r
        ridx = jnp.full((8, TI), r, jnp.int32)
        if r == 0:
            bv, bi = d, ridx
        else:
            lt = d < bv
            bv = jnp.where(lt, d, bv)
            bi = jnp.where(lt, ridx, bi)
    sub = lax.broadcasted_iota(jnp.int32, (8, TI), 0)
    bj = bi * 8 + sub
    v, jx = bv, bj
    for h in (4, 2, 1):
        v0, v1 = v[:h], v[h : 2 * h]
        j0, j1 = jx[:h], jx[h : 2 * h]
        take1 = (v1 < v0) | ((v1 == v0) & (j1 < j0))
        v = jnp.where(take1, v1, v0)
        jx = jnp.where(take1, j1, j0)
    gidx = jx + pj * TJ

    better = (v < bval[...]) | (pj == 0)
    bval[...] = jnp.where(better, v, bval[...])
    bidx[...] = jnp.where(better, gidx, bidx[...])


def _dot_tile(w_ref, j, x_ref):
    w_tile = w_ref[pl.ds(j * TJ, TJ), :]
    return lax.dot_general(
        w_tile, x_ref[...], (((1,), (1,)), ((), ())),
        preferred_element_type=jnp.float32,
        precision=lax.Precision.DEFAULT,
    )


def _argmin_body(x_ref, w_ref, xsqa_ref, wsqa_ref, xsqb_ref, wsqb_ref,
                 out_ref, mma_ref, mmb_ref, bval, bidx):
    t = pl.program_id(0)
    c = NSTEPS - 1
    ja = jnp.minimum(2 * t, c) % NJ
    jb = jnp.minimum(2 * t + 1, c) % NJ
    pja = jnp.maximum(2 * t - 1, 0) % NJ
    pjb = jnp.minimum(2 * t, c) % NJ

    mma_ref[...] = _dot_tile(w_ref, ja, x_ref)
    _epilogue(mmb_ref, xsqa_ref, wsqa_ref, pja, bval, bidx)
    out_ref[0] = bidx[...]
    mmb_ref[...] = _dot_tile(w_ref, jb, x_ref)
    _epilogue(mma_ref, xsqb_ref, wsqb_ref, pjb, bval, bidx)


def _bmu_indices(flat, emb_weight, xsq_row, wsq_col):
    c = NSTEPS - 1
    out = pl.pallas_call(
        _argmin_body,
        grid=(NSTEPS // 2 + 1,),
        in_specs=[
            pl.BlockSpec((TI, D), lambda t: (jnp.minimum(2 * t, c) // NJ, 0)),
            pl.BlockSpec((V, D), lambda t: (0, 0)),
            pl.BlockSpec((1, TI), lambda t: (0, jnp.maximum(2 * t - 1, 0) // NJ)),
            pl.BlockSpec((TJ, 1), lambda t: (jnp.maximum(2 * t - 1, 0) % NJ, 0)),
            pl.BlockSpec((1, TI), lambda t: (0, jnp.minimum(2 * t, c) // NJ)),
            pl.BlockSpec((TJ, 1), lambda t: (jnp.minimum(2 * t, c) % NJ, 0)),
        ],
        out_specs=pl.BlockSpec(
            (1, 1, TI), lambda t: (jnp.maximum(2 * t - 1, 0) // NJ, 0, 0)),
        out_shape=jax.ShapeDtypeStruct((NI, 1, TI), jnp.int32),
        scratch_shapes=[
            pltpu.VMEM((TJ, TI), jnp.float32),
            pltpu.VMEM((TJ, TI), jnp.float32),
            pltpu.VMEM((1, TI), jnp.float32),
            pltpu.VMEM((1, TI), jnp.int32),
        ],
    )(flat, emb_weight, xsq_row, wsq_col, xsq_row, wsq_col)
    return out.reshape(N)


def _sc_gather(table, idx):
    NW = 32
    b_per_w = N // NW
    mesh = plsc.VectorSubcoreMesh(core_axis_name="c", subcore_axis_name="s")

    @functools.partial(
        pl.kernel, mesh=mesh,
        out_type=jax.ShapeDtypeStruct((N, D), jnp.float32),
        scratch_types=[
            pltpu.VMEM((b_per_w,), jnp.int32),
            pltpu.VMEM((b_per_w, D), jnp.float32),
            pltpu.SemaphoreType.DMA,
        ],
    )
    def k(table_hbm, idx_hbm, out_hbm, idx_v, rows_v, sem):
        wid = lax.axis_index("s") * 2 + lax.axis_index("c")
        base = wid * b_per_w
        pltpu.sync_copy(idx_hbm.at[pl.ds(base, b_per_w)], idx_v)
        pltpu.async_copy(table_hbm.at[idx_v], rows_v, sem).wait()
        pltpu.sync_copy(rows_v, out_hbm.at[pl.ds(base, b_per_w)])

    return k(table, idx)


def kernel(input, emb_weight):
    flat = input.reshape(-1, D)
    xsq = jnp.sum(flat ** 2, axis=1) * 0.5
    wsq = jnp.sum(emb_weight ** 2, axis=1) * 0.5
    idx = _bmu_indices(flat, emb_weight, xsq.reshape(1, N), wsq.reshape(V, 1))
    out_flat = _sc_gather(emb_weight, idx)
    return out_flat.reshape(input.shape)
